# baseline (speedup 1.0000x reference)
import math

import jax
import jax.numpy as jnp
import numpy as np
from jax.experimental import pallas as pl
from jax.experimental.pallas import tpu as pltpu

D_POSITION = 1024
D_CONTENT = 4096
NUM_TILES = 64
ROWS = 1024


QSUB = 32


def _pe_factor_tables(seq):
    nfreq = D_POSITION // 2
    w = np.exp(np.arange(0, D_POSITION, 2, dtype=np.float32)
               * (-math.log(10000.0) / D_POSITION))
    nr = seq // QSUB
    q = np.arange(QSUB, dtype=np.float32)[:, None] * w[None, :]
    r = (QSUB * np.arange(nr, dtype=np.float32))[:, None] * w[None, :]
    return np.concatenate(
        [np.sin(q), np.cos(q), np.sin(r), np.cos(r)], axis=0
    ).astype(np.float32), nr, nfreq


def _dot(a, b):
    return jax.lax.dot_general(
        a, b, (((1,), (0,)), ((), ())), preferred_element_type=jnp.float32)


def _router_body(seq, nr, nfreq):
    reps = ROWS // seq

    def body(wts_ref, tab_ref, x_ref, sigab_ref, sigc_ref,
             scores_ref, idx_ref, posb_ref):
        i = pl.program_id(0)
        pw = wts_ref[0]
        cw = wts_ref[1]

        @pl.when(i == 0)
        def _():
            sq = tab_ref[0:QSUB]
            cq = tab_ref[QSUB:2 * QSUB]
            a = sigab_ref[0:nfreq]
            b = sigab_ref[nfreq:]
            for r in range(nr):
                srow = tab_ref[2 * QSUB + r, :][None, :]
                crow = tab_ref[2 * QSUB + nr + r, :][None, :]
                sin_t = cq * srow + sq * crow
                cos_t = cq * crow - sq * srow
                posb_ref[QSUB * r:QSUB * (r + 1), :] = (
                    _dot(sin_t, a) + _dot(cos_t, b))

        content = _dot(x_ref[...], sigc_ref[...])
        posb = jnp.concatenate([posb_ref[...]] * reps, axis=0)
        scores = cw * content + pw * posb
        scores_ref[...] = scores

        bits = jax.lax.bitcast_convert_type(scores, jnp.int32)
        key = bits ^ (jax.lax.shift_right_arithmetic(bits, 31) & 0x7FFFFFFF)
        iota = jax.lax.broadcasted_iota(jnp.int32, scores.shape, 1)
        packed = (key & ~jnp.int32(NUM_TILES - 1)) | (NUM_TILES - 1 - iota)
        m = jnp.max(packed, axis=-1)
        idx = (NUM_TILES - 1) - (m & (NUM_TILES - 1))
        idx_ref[...] = idx.reshape(ROWS // 128, 128)

    return body


def kernel(x, positions, signatures, position_weight, content_weight):
    del positions
    batch, seq, _ = x.shape
    rows_total = batch * seq
    n_steps = rows_total // ROWS
    tab, nr, nfreq = _pe_factor_tables(seq)
    tab = jnp.asarray(tab)
    sig_p = signatures[:, :D_POSITION]
    sig_ab = jnp.concatenate([sig_p[:, 0::2].T, sig_p[:, 1::2].T], axis=0)
    sig_con = signatures[:, D_POSITION:].T

    pw = jax.nn.sigmoid(position_weight)
    cw = jax.nn.sigmoid(content_weight)
    total = pw + cw
    wts = jnp.stack([pw / total, cw / total])

    x2 = x.reshape(rows_total, D_CONTENT)

    scores2, idx2 = pl.pallas_call(
        _router_body(seq, nr, nfreq),
        grid=(n_steps,),
        in_specs=[
            pl.BlockSpec(memory_space=pltpu.SMEM),
            pl.BlockSpec((2 * QSUB + 2 * nr, nfreq), lambda i: (0, 0)),
            pl.BlockSpec((ROWS, D_CONTENT), lambda i: (i, 0)),
            pl.BlockSpec((D_POSITION, NUM_TILES), lambda i: (0, 0)),
            pl.BlockSpec((D_CONTENT, NUM_TILES), lambda i: (0, 0)),
        ],
        out_specs=[
            pl.BlockSpec((ROWS, NUM_TILES), lambda i: (i, 0)),
            pl.BlockSpec((ROWS // 128, 128), lambda i: (i, 0)),
        ],
        out_shape=[
            jax.ShapeDtypeStruct((rows_total, NUM_TILES), jnp.float32),
            jax.ShapeDtypeStruct((rows_total // 128, 128), jnp.int32),
        ],
        scratch_shapes=[pltpu.VMEM((seq, NUM_TILES), jnp.float32)],
    )(wts, tab, x2, sig_ab, sig_con)

    scores = scores2.reshape(batch, seq, NUM_TILES)
    indices = idx2.reshape(batch, seq)
    return indices, scores


# speedup vs baseline: 1.2374x; 1.2374x over previous
import math

import jax
import jax.numpy as jnp
import numpy as np
from jax.experimental import pallas as pl
from jax.experimental.pallas import tpu as pltpu

D_POSITION = 1024
D_CONTENT = 4096
NUM_TILES = 64
ROWS = 1024


QSUB = 32


def _pe_factor_tables(seq):
    nfreq = D_POSITION // 2
    nr = seq // QSUB
    w = np.exp(np.arange(0, D_POSITION, 2, dtype=np.float32)
               * (-math.log(10000.0) / D_POSITION))

    def interleave(even, odd):
        out = np.zeros((even.shape[0], D_POSITION), dtype=np.float32)
        out[:, 0::2] = even
        out[:, 1::2] = odd
        return out

    q = np.arange(QSUB, dtype=np.float32)[:, None] * w[None, :]
    a = (QSUB * np.arange(nr, dtype=np.float32))[:, None] * w[None, :]
    cq = interleave(np.cos(q), np.cos(q))
    sq = interleave(np.sin(q), -np.sin(q))
    ar = interleave(np.sin(a), np.cos(a))
    asw = interleave(np.cos(a), np.sin(a))
    return np.concatenate([cq, sq, ar, asw], axis=0).astype(np.float32), nr


def _dot(a, b):
    return jax.lax.dot_general(
        a, b, (((1,), (0,)), ((), ())), preferred_element_type=jnp.float32)


def _router_body(seq, nr):
    reps = ROWS // seq

    def body(wts_ref, tab_ref, x_ref, sigp_ref, sigc_ref,
             scores_ref, idx_ref, posb_ref):
        i = pl.program_id(0)
        pw = wts_ref[0]
        cw = wts_ref[1]

        @pl.when(i == 0)
        def _():
            cq = tab_ref[0:QSUB]
            sq = tab_ref[QSUB:2 * QSUB]
            for r in range(nr):
                arow = tab_ref[2 * QSUB + r, :][None, :]
                asw = tab_ref[2 * QSUB + nr + r, :][None, :]
                pe_blk = arow * cq + asw * sq
                posb_ref[QSUB * r:QSUB * (r + 1), :] = jax.lax.dot_general(
                    pe_blk, sigp_ref[...], (((1,), (0,)), ((), ())),
                    preferred_element_type=jnp.float32,
                    precision=jax.lax.Precision.HIGHEST)

        content = _dot(x_ref[...], sigc_ref[...])
        posb = jnp.concatenate([posb_ref[...]] * reps, axis=0)
        scores = cw * content + pw * posb
        scores_ref[...] = scores

        bits = jax.lax.bitcast_convert_type(scores, jnp.int32)
        key = bits ^ (jax.lax.shift_right_arithmetic(bits, 31) & 0x7FFFFFFF)
        iota = jax.lax.broadcasted_iota(jnp.int32, scores.shape, 1)
        packed = (key & ~jnp.int32(NUM_TILES - 1)) | (NUM_TILES - 1 - iota)
        m = jnp.max(packed, axis=-1)
        idx = (NUM_TILES - 1) - (m & (NUM_TILES - 1))
        idx_ref[...] = idx.reshape(ROWS // 128, 128)

    return body


def kernel(x, positions, signatures, position_weight, content_weight):
    del positions
    batch, seq, _ = x.shape
    rows_total = batch * seq
    n_steps = rows_total // ROWS
    tab, nr = _pe_factor_tables(seq)
    tab = jnp.asarray(tab)
    sig_pos = signatures[:, :D_POSITION].T
    sig_con = signatures[:, D_POSITION:].T

    pw = jax.nn.sigmoid(position_weight)
    cw = jax.nn.sigmoid(content_weight)
    total = pw + cw
    wts = jnp.stack([pw / total, cw / total])

    x2 = x.reshape(rows_total, D_CONTENT)

    scores2, idx2 = pl.pallas_call(
        _router_body(seq, nr),
        grid=(n_steps,),
        in_specs=[
            pl.BlockSpec(memory_space=pltpu.SMEM),
            pl.BlockSpec((2 * QSUB + 2 * nr, D_POSITION), lambda i: (0, 0)),
            pl.BlockSpec((ROWS, D_CONTENT), lambda i: (i, 0)),
            pl.BlockSpec((D_POSITION, NUM_TILES), lambda i: (0, 0)),
            pl.BlockSpec((D_CONTENT, NUM_TILES), lambda i: (0, 0)),
        ],
        out_specs=[
            pl.BlockSpec((ROWS, NUM_TILES), lambda i: (i, 0)),
            pl.BlockSpec((ROWS // 128, 128), lambda i: (i, 0)),
        ],
        out_shape=[
            jax.ShapeDtypeStruct((rows_total, NUM_TILES), jnp.float32),
            jax.ShapeDtypeStruct((rows_total // 128, 128), jnp.int32),
        ],
        scratch_shapes=[pltpu.VMEM((seq, NUM_TILES), jnp.float32)],
    )(wts, tab, x2, sig_pos, sig_con)

    scores = scores2.reshape(batch, seq, NUM_TILES)
    indices = idx2.reshape(batch, seq)
    return indices, scores


# speedup vs baseline: 1.3289x; 1.0740x over previous
import math

import jax
import jax.numpy as jnp
import numpy as np
from jax.experimental import pallas as pl
from jax.experimental.pallas import tpu as pltpu

D_POSITION = 1024
D_CONTENT = 4096
NUM_TILES = 64
ROWS = 1024


def _sinusoidal_pe(seq_len, d_model):
    pe = np.zeros((seq_len, d_model), dtype=np.float32)
    position = np.arange(0, seq_len, dtype=np.float32)[:, None]
    div_term = np.exp(
        np.arange(0, d_model, 2, dtype=np.float32) * (-math.log(10000.0) / d_model)
    )
    pe[:, 0::2] = np.sin(position * div_term)
    pe[:, 1::2] = np.cos(position * div_term)
    return pe


def _dot(a, b):
    return jax.lax.dot_general(
        a, b, (((1,), (0,)), ((), ())), preferred_element_type=jnp.float32)


def _router_body(seq):
    reps = ROWS // seq

    def body(wts_ref, pe_ref, x_ref, sigp_ref, sigc_ref,
             scores_ref, idx_ref, posb_ref):
        i = pl.program_id(0)
        pw = wts_ref[0]
        cw = wts_ref[1]

        @pl.when(i == 0)
        def _():
            posb_ref[...] = _dot(pe_ref[...], sigp_ref[...])

        content = _dot(x_ref[...], sigc_ref[...])
        posb = jnp.concatenate([posb_ref[...]] * reps, axis=0)
        scores = cw * content + pw * posb
        scores_ref[...] = scores

        bits = jax.lax.bitcast_convert_type(scores, jnp.int32)
        key = bits ^ (jax.lax.shift_right_arithmetic(bits, 31) & 0x7FFFFFFF)
        iota = jax.lax.broadcasted_iota(jnp.int32, scores.shape, 1)
        packed = (key & ~jnp.int32(NUM_TILES - 1)) | (NUM_TILES - 1 - iota)
        m = jnp.max(packed, axis=-1)
        idx = (NUM_TILES - 1) - (m & (NUM_TILES - 1))
        idx_ref[...] = idx.reshape(ROWS // 128, 128)

    return body


def kernel(x, positions, signatures, position_weight, content_weight):
    del positions
    batch, seq, _ = x.shape
    rows_total = batch * seq
    n_steps = rows_total // ROWS
    pe = jnp.asarray(_sinusoidal_pe(seq, D_POSITION))
    sig_pos = signatures[:, :D_POSITION].T
    sig_con = signatures[:, D_POSITION:].T

    pw = jax.nn.sigmoid(position_weight)
    cw = jax.nn.sigmoid(content_weight)
    total = pw + cw
    wts = jnp.stack([pw / total, cw / total])

    x2 = x.reshape(rows_total, D_CONTENT)

    scores2, idx2 = pl.pallas_call(
        _router_body(seq),
        grid=(n_steps,),
        in_specs=[
            pl.BlockSpec(memory_space=pltpu.SMEM),
            pl.BlockSpec((seq, D_POSITION), lambda i: (0, 0)),
            pl.BlockSpec((ROWS, D_CONTENT), lambda i: (i, 0)),
            pl.BlockSpec((D_POSITION, NUM_TILES), lambda i: (0, 0)),
            pl.BlockSpec((D_CONTENT, NUM_TILES), lambda i: (0, 0)),
        ],
        out_specs=[
            pl.BlockSpec((ROWS, NUM_TILES), lambda i: (i, 0)),
            pl.BlockSpec((ROWS // 128, 128), lambda i: (i, 0)),
        ],
        out_shape=[
            jax.ShapeDtypeStruct((rows_total, NUM_TILES), jnp.float32),
            jax.ShapeDtypeStruct((rows_total // 128, 128), jnp.int32),
        ],
        scratch_shapes=[pltpu.VMEM((seq, NUM_TILES), jnp.float32)],
    )(wts, pe, x2, sig_pos, sig_con)

    scores = scores2.reshape(batch, seq, NUM_TILES)
    indices = idx2.reshape(batch, seq)
    return indices, scores
